# baseline (speedup 1.0000x reference)
import functools

import jax
import jax.numpy as jnp
from jax import lax
from jax.experimental import pallas as pl
from jax.experimental.pallas import tpu as pltpu
from jax.experimental.pallas import tpu_sc as plsc

NN = 10000
DD = 128
NB = 10240
NC, NS = 2, 16
NW = NC * NS
CH = 128
NBUF = 2
SEGA = 40
CPS = 160
CPT = CPS // NC
EP = NS * CPS * CH
JUNK = NB - NN
RPT = NB // NS
SEG = 16
SEGD = 8
EPS_BN = 1e-5


@functools.lru_cache(maxsize=None)
def _mesh():
    return plsc.VectorSubcoreMesh(core_axis_name="c", subcore_axis_name="s",
                                  num_cores=NC, num_subcores=NS)


@functools.lru_cache(maxsize=None)
def _mesh1():
    return plsc.VectorSubcoreMesh(core_axis_name="c", subcore_axis_name="s",
                                  num_cores=1, num_subcores=NS)


def _deg_body(srcs_hbm, dsts_hbm, out_hbm, sseg_v, dseg_v, histf_v):
    c = lax.axis_index("c")
    t = lax.axis_index("s")
    slab = c * NS + t

    zeros16 = jnp.zeros((16,), jnp.float32)
    ones16 = jnp.ones((16,), jnp.float32)

    @pl.loop(0, 2 * NB // 128)
    def _(r):
        for k in range(8):
            histf_v[pl.ds(r * 128 + k * 16, 16)] = zeros16

    pltpu.sync_copy(srcs_hbm.at[t, pl.ds(c * CPT, CPT)], sseg_v)
    pltpu.sync_copy(dsts_hbm.at[t, pl.ds(c * CPT, CPT)], dseg_v)

    @pl.loop(0, CPT)
    def _(ci):
        for k8 in range(8):
            si = sseg_v[ci, pl.ds(k8 * 16, 16)]
            plsc.addupdate_scatter(histf_v, [si], ones16)
            di = dseg_v[ci, pl.ds(k8 * 16, 16)]
            plsc.addupdate_scatter(histf_v, [NB + di], ones16)

    pltpu.sync_copy(histf_v, out_hbm.at[pl.ds(slab * 2 * NB, 2 * NB)])


@functools.lru_cache(maxsize=None)
def _deg_call():
    return pl.kernel(
        _deg_body,
        out_type=jax.ShapeDtypeStruct((NW * 2 * NB,), jnp.float32),
        mesh=_mesh(),
        scratch_types=[
            pltpu.VMEM((CPT, CH), jnp.int32),
            pltpu.VMEM((CPT, CH), jnp.int32),
            pltpu.VMEM((2 * NB,), jnp.float32),
        ],
        compiler_params=pltpu.CompilerParams(needs_layout_passes=False),
    )


def _agg_body(s_hbm, srcs_hbm, dsts_hbm, out_hbm, sidx_v, didx_v, bufs_v,
              acc_sh, *sems):
    c = lax.axis_index("c")
    t = lax.axis_index("s")

    zeros16 = jnp.zeros((16,), jnp.float32)

    @pl.loop(0, CH)
    def _(r):
        for k in range(DD // 16):
            bufs_v[0, r, pl.ds(k * 16, 16)] = zeros16
    for k in range(RPT // CH):
        pltpu.sync_copy(bufs_v.at[0], acc_sh.at[pl.ds(t * RPT + k * CH, CH)])
    plsc.subcore_barrier()

    @pl.loop(0, CPT // SEGA)
    def _(g):
        base = c * CPT + g * SEGA
        pltpu.sync_copy(srcs_hbm.at[t, pl.ds(base, SEGA)], sidx_v)
        pltpu.sync_copy(dsts_hbm.at[t, pl.ds(base, SEGA)], didx_v)
        for b in range(NBUF):
            pltpu.async_copy(s_hbm.at[sidx_v.at[b]], bufs_v.at[b], sems[b])
        for jj in range(SEGA):
            b = jj % NBUF
            pltpu.make_async_copy(
                s_hbm.at[sidx_v.at[jj]], bufs_v.at[b], sems[b]).wait()
            pltpu.sync_copy(bufs_v.at[b], acc_sh.at[didx_v.at[jj]], add=True)
            if jj + NBUF < SEGA:
                pltpu.async_copy(
                    s_hbm.at[sidx_v.at[jj + NBUF]], bufs_v.at[b], sems[b])

    plsc.subcore_barrier()
    for k in range(RPT // CH):
        pltpu.sync_copy(acc_sh.at[pl.ds(t * RPT + k * CH, CH)],
                        out_hbm.at[pl.ds(c * NB + t * RPT + k * CH, CH)])


@functools.lru_cache(maxsize=None)
def _agg_call():
    return pl.kernel(
        _agg_body,
        out_type=jax.ShapeDtypeStruct((2 * NB, DD), jnp.float32),
        mesh=_mesh(),
        scratch_types=[
            pltpu.VMEM((SEGA, CH), jnp.int32),
            pltpu.VMEM((SEGA, CH), jnp.int32),
            pltpu.VMEM((NBUF, CH, DD), jnp.float32),
            pltpu.VMEM_SHARED((NB, DD), jnp.float32),
        ] + [pltpu.SemaphoreType.DMA] * NBUF,
        compiler_params=pltpu.CompilerParams(needs_layout_passes=False),
    )


def _prescale_body(x_ref, dT_ref, s_ref):
    dsrc = lax.rsqrt(jnp.maximum(dT_ref[0:NN, 0:1], 1.0))
    s_ref[0:NN, :] = x_ref[...] * dsrc
    s_ref[NN:NB, :] = jnp.zeros((NB - NN, DD), jnp.float32)


_prescale_call = pl.pallas_call(
    _prescale_body,
    out_shape=jax.ShapeDtypeStruct((NB, DD), jnp.float32),
)


def _dot16(a, b):
    return jnp.dot(a.astype(jnp.bfloat16), b.astype(jnp.bfloat16),
                   preferred_element_type=jnp.float32)


def _bn_head(h, W1, b1, g, be, W2, b2):
    y = _dot16(h, W1) + b1
    mu = jnp.mean(y, axis=0, keepdims=True)
    var = jnp.mean((y - mu) ** 2, axis=0, keepdims=True)
    y = (y - mu) / jnp.sqrt(var + EPS_BN) * g + be
    y = jnp.maximum(y, 0.0)
    return _dot16(y, W2) + b2


def _dense_body(p_ref, dT_ref, W_ref, b_ref, Wp1_ref, bp1_ref, gp1_ref,
                bep1_ref, Wp2_ref, bp2_ref, Wq1_ref, bq1_ref, gq1_ref,
                beq1_ref, Wq2_ref, bq2_ref, out_ref):
    ddst = lax.rsqrt(jnp.maximum(dT_ref[:NN, 1:2], 1.0))
    agg = (p_ref[0:NN, :] + p_ref[NB:NB + NN, :]) * ddst
    h = jnp.maximum(_dot16(agg, W_ref[...]) + b_ref[...], 0.0)
    z = _bn_head(h, Wp1_ref[...], bp1_ref[...], gp1_ref[...], bep1_ref[...],
                 Wp2_ref[...], bp2_ref[...])
    pred = _bn_head(z, Wq1_ref[...], bq1_ref[...], gq1_ref[...], beq1_ref[...],
                    Wq2_ref[...], bq2_ref[...])
    nx = jnp.sqrt(jnp.sum(pred * pred, axis=1, keepdims=True))
    ny = jnp.sqrt(jnp.sum(z * z, axis=1, keepdims=True))
    xn = pred / jnp.maximum(nx, 1e-12)
    yn = z / jnp.maximum(ny, 1e-12)
    ip = jnp.sum(xn * yn, axis=1, keepdims=True)
    loss = 2.0 - 2.0 * jnp.sum(ip) / NN
    out_ref[...] = jnp.broadcast_to(loss, (1, 128))


_dense_call = pl.pallas_call(
    _dense_body,
    out_shape=jax.ShapeDtypeStruct((1, 128), jnp.float32),
)


def kernel(x, edge_index, W_gcn, b_gcn, Wp1, bp1, gp1, bep1, Wp2, bp2,
           Wq1, bq1, gq1, beq1, Wq2, bq2):
    E = edge_index.shape[1]
    pad = EP - E
    ar = jnp.arange(pad, dtype=jnp.int32)
    fill_s = NN + ar % JUNK
    fill_d = NN + (ar * 7 + 3) % JUNK
    srcs = jnp.concatenate([edge_index[0], fill_s]).reshape(NS, CPS, CH)
    dsts = jnp.concatenate([edge_index[1], fill_d]).reshape(NS, CPS, CH)

    hists = _deg_call()(srcs, dsts)
    dT = jnp.transpose(hists.reshape(NW, 2, NB).sum(axis=0))
    s = _prescale_call(x, dT)
    p = _agg_call()(s, srcs, dsts)
    loss = _dense_call(p, dT, W_gcn, b_gcn.reshape(1, DD),
                       Wp1, bp1.reshape(1, DD), gp1.reshape(1, DD),
                       bep1.reshape(1, DD), Wp2, bp2.reshape(1, DD),
                       Wq1, bq1.reshape(1, DD), gq1.reshape(1, DD),
                       beq1.reshape(1, DD), Wq2, bq2.reshape(1, DD))
    return loss[0, 0]


# speedup vs baseline: 1.0344x; 1.0344x over previous
import functools

import jax
import jax.numpy as jnp
from jax import lax
from jax.experimental import pallas as pl
from jax.experimental.pallas import tpu as pltpu
from jax.experimental.pallas import tpu_sc as plsc

NN = 10000
DD = 128
NB = 10240
NC, NS = 2, 16
NW = NC * NS
CH = 128
NBUF = 2
SEGA = 40
CPS = 160
CPT = CPS // NC
EP = NS * CPS * CH
JUNK = NB - NN
RPT = NB // NS
SEG = 16
SEGD = 8
EPS_BN = 1e-5


@functools.lru_cache(maxsize=None)
def _mesh():
    return plsc.VectorSubcoreMesh(core_axis_name="c", subcore_axis_name="s",
                                  num_cores=NC, num_subcores=NS)


@functools.lru_cache(maxsize=None)
def _mesh1():
    return plsc.VectorSubcoreMesh(core_axis_name="c", subcore_axis_name="s",
                                  num_cores=1, num_subcores=NS)


def _deg_body(srcs_hbm, dsts_hbm, out_hbm, sseg_v, dseg_v, histf_v):
    c = lax.axis_index("c")
    t = lax.axis_index("s")
    slab = c * NS + t

    zeros16 = jnp.zeros((16,), jnp.float32)
    ones16 = jnp.ones((16,), jnp.float32)

    @pl.loop(0, 2 * NB // 128)
    def _(r):
        for k in range(8):
            histf_v[pl.ds(r * 128 + k * 16, 16)] = zeros16

    pltpu.sync_copy(srcs_hbm.at[t, pl.ds(c * CPT, CPT)], sseg_v)
    pltpu.sync_copy(dsts_hbm.at[t, pl.ds(c * CPT, CPT)], dseg_v)

    @pl.loop(0, CPT)
    def _(ci):
        for k8 in range(8):
            si = sseg_v[ci, pl.ds(k8 * 16, 16)]
            plsc.addupdate_scatter(histf_v, [si], ones16)
            di = dseg_v[ci, pl.ds(k8 * 16, 16)]
            plsc.addupdate_scatter(histf_v, [NB + di], ones16)

    pltpu.sync_copy(histf_v, out_hbm.at[pl.ds(slab * 2 * NB, 2 * NB)])


@functools.lru_cache(maxsize=None)
def _deg_call():
    return pl.kernel(
        _deg_body,
        out_type=jax.ShapeDtypeStruct((NW * 2 * NB,), jnp.float32),
        mesh=_mesh(),
        scratch_types=[
            pltpu.VMEM((CPT, CH), jnp.int32),
            pltpu.VMEM((CPT, CH), jnp.int32),
            pltpu.VMEM((2 * NB,), jnp.float32),
        ],
        compiler_params=pltpu.CompilerParams(needs_layout_passes=False),
    )


def _agg_body(s_hbm, srcs_hbm, dsts_hbm, out_hbm, sidx_v, didx_v, bufs_v,
              acc_sh, *sems):
    c = lax.axis_index("c")
    t = lax.axis_index("s")

    zeros16 = jnp.zeros((16,), jnp.float32)

    @pl.loop(0, CH)
    def _(r):
        for k in range(DD // 16):
            bufs_v[0, r, pl.ds(k * 16, 16)] = zeros16
    for k in range(RPT // CH):
        pltpu.sync_copy(bufs_v.at[0], acc_sh.at[pl.ds(t * RPT + k * CH, CH)])
    plsc.subcore_barrier()

    @pl.loop(0, CPT // SEGA)
    def _(g):
        base = c * CPT + g * SEGA
        pltpu.sync_copy(srcs_hbm.at[t, pl.ds(base, SEGA)], sidx_v)
        pltpu.sync_copy(dsts_hbm.at[t, pl.ds(base, SEGA)], didx_v)
        for b in range(NBUF):
            pltpu.async_copy(s_hbm.at[sidx_v.at[b]], bufs_v.at[b], sems[b])
        for jj in range(SEGA):
            b = jj % NBUF
            pltpu.make_async_copy(
                s_hbm.at[sidx_v.at[jj]], bufs_v.at[b], sems[b]).wait()
            pltpu.sync_copy(bufs_v.at[b], acc_sh.at[didx_v.at[jj]], add=True)
            if jj + NBUF < SEGA:
                pltpu.async_copy(
                    s_hbm.at[sidx_v.at[jj + NBUF]], bufs_v.at[b], sems[b])

    plsc.subcore_barrier()
    for k in range(RPT // CH):
        pltpu.sync_copy(acc_sh.at[pl.ds(t * RPT + k * CH, CH)],
                        out_hbm.at[pl.ds(c * NB + t * RPT + k * CH, CH)])


@functools.lru_cache(maxsize=None)
def _agg_call():
    return pl.kernel(
        _agg_body,
        out_type=jax.ShapeDtypeStruct((2 * NB, DD), jnp.float32),
        mesh=_mesh(),
        scratch_types=[
            pltpu.VMEM((SEGA, CH), jnp.int32),
            pltpu.VMEM((SEGA, CH), jnp.int32),
            pltpu.VMEM((NBUF, CH, DD), jnp.float32),
            pltpu.VMEM_SHARED((NB, DD), jnp.float32),
        ] + [pltpu.SemaphoreType.DMA] * NBUF,
        compiler_params=pltpu.CompilerParams(needs_layout_passes=False),
    )


def _prescale_body(x_ref, h_ref, s_ref):
    rsum = jnp.sum(h_ref[...], axis=0, keepdims=True)
    dsrc = jnp.transpose(
        lax.rsqrt(jnp.maximum(rsum[:, 0:NN], 1.0)))
    s_ref[0:NN, :] = x_ref[...] * dsrc
    s_ref[NN:NB, :] = jnp.zeros((NB - NN, DD), jnp.float32)


_prescale_call = pl.pallas_call(
    _prescale_body,
    out_shape=jax.ShapeDtypeStruct((NB, DD), jnp.float32),
)


def _bn_head(h, W1, b1, g, be, W2, b2):
    y = jnp.dot(h, W1, preferred_element_type=jnp.float32) + b1
    mu = jnp.mean(y, axis=0, keepdims=True)
    var = jnp.mean((y - mu) ** 2, axis=0, keepdims=True)
    y = (y - mu) / jnp.sqrt(var + EPS_BN) * g + be
    y = jnp.maximum(y, 0.0)
    return jnp.dot(y, W2, preferred_element_type=jnp.float32) + b2


def _dense_body(p_ref, dT_ref, W_ref, b_ref, Wp1_ref, bp1_ref, gp1_ref,
                bep1_ref, Wp2_ref, bp2_ref, Wq1_ref, bq1_ref, gq1_ref,
                beq1_ref, Wq2_ref, bq2_ref, out_ref):
    rsum = jnp.sum(dT_ref[...], axis=0, keepdims=True)
    ddst = jnp.transpose(
        lax.rsqrt(jnp.maximum(rsum[:, NB:NB + NN], 1.0)))
    agg = (p_ref[0:NN, :] + p_ref[NB:NB + NN, :]) * ddst
    h = jnp.maximum(
        jnp.dot(agg, W_ref[...], preferred_element_type=jnp.float32)
        + b_ref[...], 0.0)
    z = _bn_head(h, Wp1_ref[...], bp1_ref[...], gp1_ref[...], bep1_ref[...],
                 Wp2_ref[...], bp2_ref[...])
    pred = _bn_head(z, Wq1_ref[...], bq1_ref[...], gq1_ref[...], beq1_ref[...],
                    Wq2_ref[...], bq2_ref[...])
    nx = jnp.sqrt(jnp.sum(pred * pred, axis=1, keepdims=True))
    ny = jnp.sqrt(jnp.sum(z * z, axis=1, keepdims=True))
    xn = pred / jnp.maximum(nx, 1e-12)
    yn = z / jnp.maximum(ny, 1e-12)
    ip = jnp.sum(xn * yn, axis=1, keepdims=True)
    loss = 2.0 - 2.0 * jnp.sum(ip) / NN
    out_ref[...] = jnp.broadcast_to(loss, (1, 128))


_dense_call = pl.pallas_call(
    _dense_body,
    out_shape=jax.ShapeDtypeStruct((1, 128), jnp.float32),
)


def kernel(x, edge_index, W_gcn, b_gcn, Wp1, bp1, gp1, bep1, Wp2, bp2,
           Wq1, bq1, gq1, beq1, Wq2, bq2):
    E = edge_index.shape[1]
    pad = EP - E
    ar = jnp.arange(pad, dtype=jnp.int32)
    fill_s = NN + ar % JUNK
    fill_d = NN + (ar * 7 + 3) % JUNK
    srcs = jnp.concatenate([edge_index[0], fill_s]).reshape(NS, CPS, CH)
    dsts = jnp.concatenate([edge_index[1], fill_d]).reshape(NS, CPS, CH)

    hists = _deg_call()(srcs, dsts).reshape(NW, 2 * NB)
    s = _prescale_call(x, hists)
    p = _agg_call()(s, srcs, dsts)
    loss = _dense_call(p, hists, W_gcn, b_gcn.reshape(1, DD),
                       Wp1, bp1.reshape(1, DD), gp1.reshape(1, DD),
                       bep1.reshape(1, DD), Wp2, bp2.reshape(1, DD),
                       Wq1, bq1.reshape(1, DD), gq1.reshape(1, DD),
                       beq1.reshape(1, DD), Wq2, bq2.reshape(1, DD))
    return loss[0, 0]


# speedup vs baseline: 1.0346x; 1.0002x over previous
import functools

import jax
import jax.numpy as jnp
from jax import lax
from jax.experimental import pallas as pl
from jax.experimental.pallas import tpu as pltpu
from jax.experimental.pallas import tpu_sc as plsc

NN = 10000
DD = 128
NB = 10240
NC, NS = 2, 16
NW = NC * NS
CH = 128
NBUF = 2
SEGA = 40
CPS = 160
CPT = CPS // NC
EP = NS * CPS * CH
JUNK = NB - NN
RPT = NB // NS
EPS_BN = 1e-5


@functools.lru_cache(maxsize=None)
def _mesh():
    return plsc.VectorSubcoreMesh(core_axis_name="c", subcore_axis_name="s",
                                  num_cores=NC, num_subcores=NS)


@functools.lru_cache(maxsize=None)
def _mesh1():
    return plsc.VectorSubcoreMesh(core_axis_name="c", subcore_axis_name="s",
                                  num_cores=1, num_subcores=NS)


def _deg_body(srcs_hbm, dsts_hbm, out_hbm, sseg_v, dseg_v, histf_v):
    c = lax.axis_index("c")
    t = lax.axis_index("s")
    slab = c * NS + t

    zeros16 = jnp.zeros((16,), jnp.float32)
    ones16 = jnp.ones((16,), jnp.float32)

    @pl.loop(0, 2 * NB // 128)
    def _(r):
        for k in range(8):
            histf_v[pl.ds(r * 128 + k * 16, 16)] = zeros16

    pltpu.sync_copy(srcs_hbm.at[t, pl.ds(c * CPT, CPT)], sseg_v)
    pltpu.sync_copy(dsts_hbm.at[t, pl.ds(c * CPT, CPT)], dseg_v)

    @pl.loop(0, CPT)
    def _(ci):
        for k8 in range(8):
            si = sseg_v[ci, pl.ds(k8 * 16, 16)]
            plsc.addupdate_scatter(histf_v, [si], ones16)
            di = dseg_v[ci, pl.ds(k8 * 16, 16)]
            plsc.addupdate_scatter(histf_v, [NB + di], ones16)

    pltpu.sync_copy(histf_v, out_hbm.at[pl.ds(slab * 2 * NB, 2 * NB)])


@functools.lru_cache(maxsize=None)
def _deg_call():
    return pl.kernel(
        _deg_body,
        out_type=jax.ShapeDtypeStruct((NW * 2 * NB,), jnp.float32),
        mesh=_mesh(),
        scratch_types=[
            pltpu.VMEM((CPT, CH), jnp.int32),
            pltpu.VMEM((CPT, CH), jnp.int32),
            pltpu.VMEM((2 * NB,), jnp.float32),
        ],
        compiler_params=pltpu.CompilerParams(needs_layout_passes=False),
    )


def _agg_body(s_hbm, srcs_hbm, dsts_hbm, out_hbm, sidx_v, didx_v, bufs_v,
              acc_sh, *sems):
    c = lax.axis_index("c")
    t = lax.axis_index("s")

    zeros16 = jnp.zeros((16,), jnp.float32)

    @pl.loop(0, CH)
    def _(r):
        for k in range(DD // 16):
            bufs_v[0, r, pl.ds(k * 16, 16)] = zeros16
    for k in range(RPT // CH):
        pltpu.sync_copy(bufs_v.at[0], acc_sh.at[pl.ds(t * RPT + k * CH, CH)])
    plsc.subcore_barrier()

    @pl.loop(0, CPT // SEGA)
    def _(g):
        base = c * CPT + g * SEGA
        pltpu.sync_copy(srcs_hbm.at[t, pl.ds(base, SEGA)], sidx_v)
        pltpu.sync_copy(dsts_hbm.at[t, pl.ds(base, SEGA)], didx_v)
        for b in range(NBUF):
            pltpu.async_copy(s_hbm.at[sidx_v.at[b]], bufs_v.at[b], sems[b])
        for jj in range(SEGA):
            b = jj % NBUF
            pltpu.make_async_copy(
                s_hbm.at[sidx_v.at[jj]], bufs_v.at[b], sems[b]).wait()
            pltpu.sync_copy(bufs_v.at[b], acc_sh.at[didx_v.at[jj]], add=True)
            if jj + NBUF < SEGA:
                pltpu.async_copy(
                    s_hbm.at[sidx_v.at[jj + NBUF]], bufs_v.at[b], sems[b])

    plsc.subcore_barrier()
    for k in range(RPT // CH):
        pltpu.sync_copy(acc_sh.at[pl.ds(t * RPT + k * CH, CH)],
                        out_hbm.at[pl.ds(c * NB + t * RPT + k * CH, CH)])


@functools.lru_cache(maxsize=None)
def _agg_call():
    return pl.kernel(
        _agg_body,
        out_type=jax.ShapeDtypeStruct((2 * NB, DD), jnp.float32),
        mesh=_mesh(),
        scratch_types=[
            pltpu.VMEM((SEGA, CH), jnp.int32),
            pltpu.VMEM((SEGA, CH), jnp.int32),
            pltpu.VMEM((NBUF, CH, DD), jnp.float32),
            pltpu.VMEM_SHARED((NB, DD), jnp.float32),
        ] + [pltpu.SemaphoreType.DMA] * NBUF,
        compiler_params=pltpu.CompilerParams(needs_layout_passes=False),
    )


def _prescale_body(x_ref, h_ref, s_ref):
    rsum = jnp.sum(h_ref[...], axis=0, keepdims=True)
    dsrc = jnp.transpose(
        lax.rsqrt(jnp.maximum(rsum[:, 0:NN], 1.0)))
    s_ref[0:NN, :] = x_ref[...] * dsrc
    s_ref[NN:NB, :] = jnp.zeros((NB - NN, DD), jnp.float32)


_prescale_call = pl.pallas_call(
    _prescale_body,
    out_shape=jax.ShapeDtypeStruct((NB, DD), jnp.float32),
)


def _bn_head(h, W1, b1, g, be, W2, b2):
    y = jnp.dot(h, W1, preferred_element_type=jnp.float32) + b1
    mu = jnp.mean(y, axis=0, keepdims=True)
    var = jnp.mean((y - mu) ** 2, axis=0, keepdims=True)
    y = (y - mu) / jnp.sqrt(var + EPS_BN) * g + be
    y = jnp.maximum(y, 0.0)
    return jnp.dot(y, W2, preferred_element_type=jnp.float32) + b2


def _dense_body(p_ref, dT_ref, W_ref, b_ref, Wp1_ref, bp1_ref, gp1_ref,
                bep1_ref, Wp2_ref, bp2_ref, Wq1_ref, bq1_ref, gq1_ref,
                beq1_ref, Wq2_ref, bq2_ref, out_ref):
    rsum = jnp.sum(dT_ref[...], axis=0, keepdims=True)
    ddst = jnp.transpose(
        lax.rsqrt(jnp.maximum(rsum[:, NB:NB + NN], 1.0)))
    agg = (p_ref[0:NN, :] + p_ref[NB:NB + NN, :]) * ddst
    h = jnp.maximum(
        jnp.dot(agg, W_ref[...], preferred_element_type=jnp.float32)
        + b_ref[...], 0.0)
    z = _bn_head(h, Wp1_ref[...], bp1_ref[...], gp1_ref[...], bep1_ref[...],
                 Wp2_ref[...], bp2_ref[...])
    pred = _bn_head(z, Wq1_ref[...], bq1_ref[...], gq1_ref[...], beq1_ref[...],
                    Wq2_ref[...], bq2_ref[...])
    nx = jnp.sqrt(jnp.sum(pred * pred, axis=1, keepdims=True))
    ny = jnp.sqrt(jnp.sum(z * z, axis=1, keepdims=True))
    xn = pred / jnp.maximum(nx, 1e-12)
    yn = z / jnp.maximum(ny, 1e-12)
    ip = jnp.sum(xn * yn, axis=1, keepdims=True)
    loss = 2.0 - 2.0 * jnp.sum(ip) / NN
    out_ref[...] = jnp.broadcast_to(loss, (1, 128))


_dense_call = pl.pallas_call(
    _dense_body,
    out_shape=jax.ShapeDtypeStruct((1, 128), jnp.float32),
)


def kernel(x, edge_index, W_gcn, b_gcn, Wp1, bp1, gp1, bep1, Wp2, bp2,
           Wq1, bq1, gq1, beq1, Wq2, bq2):
    E = edge_index.shape[1]
    pad = EP - E
    ar = jnp.arange(pad, dtype=jnp.int32)
    fill_s = NN + ar % JUNK
    fill_d = NN + (ar * 7 + 3) % JUNK
    srcs = jnp.concatenate([edge_index[0], fill_s]).reshape(NS, CPS, CH)
    dsts = jnp.concatenate([edge_index[1], fill_d]).reshape(NS, CPS, CH)

    hists = _deg_call()(srcs, dsts).reshape(NW, 2 * NB)
    s = _prescale_call(x, hists)
    p = _agg_call()(s, srcs, dsts)
    loss = _dense_call(p, hists, W_gcn, b_gcn.reshape(1, DD),
                       Wp1, bp1.reshape(1, DD), gp1.reshape(1, DD),
                       bep1.reshape(1, DD), Wp2, bp2.reshape(1, DD),
                       Wq1, bq1.reshape(1, DD), gq1.reshape(1, DD),
                       beq1.reshape(1, DD), Wq2, bq2.reshape(1, DD))
    return loss[0, 0]
